# baseline (speedup 1.0000x reference)
import jax
import jax.numpy as jnp
from jax import lax
from jax.experimental import pallas as pl
from jax.experimental.pallas import tpu as pltpu

N_TOK = 20000
N_CLS = 92
K = 64
BLK = 2048
NB = (N_TOK + BLK - 1) // BLK
D_H = 256
D_B = 4


def _body(logits_blk, h_any, boxes_any, logits_any,
          out_h, out_b, out_l,
          scores, idxs, sem_h, sem_b, sem_l):
    i = pl.program_id(0)

    @pl.when(i < NB)
    def _phase1():
        x = logits_blk[...]
        sc = jnp.max(x[:, : N_CLS - 1], axis=1)
        tok = i * BLK + lax.broadcasted_iota(jnp.int32, (BLK,), 0)
        sc = jnp.where(tok < N_TOK, sc, -jnp.inf)
        scores[i, :] = sc

    @pl.when(i == NB)
    def _phase2():
        flat = (lax.broadcasted_iota(jnp.int32, (NB, BLK), 0) * BLK
                + lax.broadcasted_iota(jnp.int32, (NB, BLK), 1))

        def topk_body(k, x):
            m = jnp.max(x)
            idx = jnp.min(jnp.where(x == m, flat, jnp.int32(1 << 30)))
            idxs[k] = idx
            return jnp.where(flat == idx, -jnp.inf, x)

        lax.fori_loop(0, K, topk_body, scores[...], unroll=False)

        def gather_start(k, _):
            idx = idxs[k]
            pltpu.make_async_copy(
                h_any.at[pl.ds(idx, 1), :], out_h.at[pl.ds(k, 1), :],
                sem_h).start()
            pltpu.make_async_copy(
                boxes_any.at[pl.ds(idx, 1), :], out_b.at[pl.ds(k, 1), :],
                sem_b).start()
            pltpu.make_async_copy(
                logits_any.at[pl.ds(idx, 1), :], out_l.at[pl.ds(k, 1), :],
                sem_l).start()
            return 0

        lax.fori_loop(0, K, gather_start, 0, unroll=False)

        def gather_wait(k, _):
            idx = idxs[k]
            pltpu.make_async_copy(
                h_any.at[pl.ds(idx, 1), :], out_h.at[pl.ds(k, 1), :],
                sem_h).wait()
            pltpu.make_async_copy(
                boxes_any.at[pl.ds(idx, 1), :], out_b.at[pl.ds(k, 1), :],
                sem_b).wait()
            pltpu.make_async_copy(
                logits_any.at[pl.ds(idx, 1), :], out_l.at[pl.ds(k, 1), :],
                sem_l).wait()
            return 0

        lax.fori_loop(0, K, gather_wait, 0, unroll=False)


def kernel(h, pred_boxes, pred_logits):
    h2 = h[0]
    b2 = pred_boxes[0]
    l2 = pred_logits[0]

    out_h, out_b, out_l = pl.pallas_call(
        _body,
        grid=(NB + 1,),
        in_specs=[
            pl.BlockSpec((BLK, N_CLS), lambda i: (jnp.minimum(i, NB - 1), 0)),
            pl.BlockSpec(memory_space=pl.ANY),
            pl.BlockSpec(memory_space=pl.ANY),
            pl.BlockSpec(memory_space=pl.ANY),
        ],
        out_specs=[
            pl.BlockSpec((K, D_H), lambda i: (0, 0)),
            pl.BlockSpec((K, D_B), lambda i: (0, 0)),
            pl.BlockSpec((K, N_CLS), lambda i: (0, 0)),
        ],
        out_shape=[
            jax.ShapeDtypeStruct((K, D_H), jnp.float32),
            jax.ShapeDtypeStruct((K, D_B), jnp.float32),
            jax.ShapeDtypeStruct((K, N_CLS), jnp.float32),
        ],
        scratch_shapes=[
            pltpu.VMEM((NB, BLK), jnp.float32),
            pltpu.SMEM((K,), jnp.int32),
            pltpu.SemaphoreType.DMA,
            pltpu.SemaphoreType.DMA,
            pltpu.SemaphoreType.DMA,
        ],
        compiler_params=pltpu.CompilerParams(
            dimension_semantics=("arbitrary",),
        ),
    )(l2, h2, b2, l2)

    seq = jnp.concatenate([out_h, out_b, out_l], axis=-1)[None]
    return seq
